# Initial kernel scaffold; baseline (speedup 1.0000x reference)
import jax
import jax.numpy as jnp
from jax.experimental import pallas as pl


def kernel(X, adj_indices, W0, b0, W1, b1, W2, b2):
    raise NotImplementedError("write your pallas kernel here")


# speedup vs baseline: 23.8697x; 23.8697x over previous
import functools

import jax
import jax.numpy as jnp
from jax import lax
from jax.experimental import pallas as pl
from jax.experimental.pallas import tpu as pltpu
from jax.experimental.pallas import tpu_sc as plsc

N_NODES = 10000
D = 128
NC = 2
NS = 16
NW = NC * NS

NP = 10240
HALF = NP // 2
ACC_ROWS = 5760
DUMP = 5632
ROWS_PER_TILE = ACC_ROWS // NS
ZCHUNK = 24

K_EDGE = 128

_SC_MESH = dict(core_axis_name="c", subcore_axis_name="s",
                num_cores=NC, num_subcores=NS)


def _make_deg_kernel(E):
    P = E // NW

    @functools.partial(
        pl.kernel,
        out_type=jax.ShapeDtypeStruct((NW * N_NODES,), jnp.float32),
        mesh=plsc.VectorSubcoreMesh(**_SC_MESH),
        compiler_params=pltpu.CompilerParams(needs_layout_passes=False),
        scratch_types=[
            pltpu.VMEM((P,), jnp.int32),
            pltpu.VMEM((N_NODES,), jnp.float32),
        ],
    )
    def deg_kernel(dst_hbm, out_hbm, idx_v, acc_v):
        cid = lax.axis_index("c")
        sid = lax.axis_index("s")
        wid = cid * NS + sid
        pltpu.sync_copy(dst_hbm.at[pl.ds(wid * P, P)], idx_v)

        def zero_body(i, carry):
            acc_v[pl.ds(i * 16, 16)] = jnp.zeros((16,), jnp.float32)
            return carry
        lax.fori_loop(0, N_NODES // 16, zero_body, 0)

        ones = jnp.ones((16,), jnp.float32)

        def add_body(j, carry):
            idx = idx_v[pl.ds(j * 16, 16)]
            plsc.addupdate_scatter(acc_v, [idx], ones)
            return carry
        lax.fori_loop(0, P // 16, add_body, 0)

        pltpu.sync_copy(acc_v, out_hbm.at[pl.ds(wid * N_NODES, N_NODES)])

    return deg_kernel


def _make_scatter_kernel(E):
    P2 = E // NS
    K = K_EDGE
    NV = P2 // 16
    BUF = ((P2 + 127) // 128 + 1) * 128

    @functools.partial(
        pl.kernel,
        out_type=jax.ShapeDtypeStruct((NC, ACC_ROWS, D), jnp.float32),
        mesh=plsc.VectorSubcoreMesh(**_SC_MESH),
        compiler_params=pltpu.CompilerParams(needs_layout_passes=False),
        scratch_types=[
            pltpu.VMEM((BUF,), jnp.int32),
            pltpu.VMEM((BUF,), jnp.int32),
            pltpu.VMEM((2, K, D), jnp.float32),
            pltpu.VMEM((ZCHUNK, D), jnp.float32),
            pltpu.VMEM_SHARED((ACC_ROWS, D), jnp.float32),
            pltpu.SemaphoreType.DMA,
        ],
    )
    def scat_kernel(g_hbm, src_hbm, dst_hbm, out_hbm,
                    src_v, dst_v, rows_v, zbuf_v, acc, sem):
        cid = lax.axis_index("c")
        sid = lax.axis_index("s")
        base = cid * HALF

        pltpu.sync_copy(src_hbm.at[pl.ds(sid * P2, P2)], src_v.at[pl.ds(0, P2)])
        pltpu.sync_copy(dst_hbm.at[pl.ds(sid * P2, P2)], dst_v.at[pl.ds(0, P2)])

        def zb(i, carry):
            for cc in range(D // 16):
                zbuf_v[i, pl.ds(cc * 16, 16)] = jnp.zeros((16,), jnp.float32)
            return carry
        lax.fori_loop(0, ZCHUNK, zb, 0)
        for t in range(ROWS_PER_TILE // ZCHUNK):
            pltpu.sync_copy(
                zbuf_v, acc.at[pl.ds(sid * ROWS_PER_TILE + t * ZCHUNK, ZCHUNK)])

        def comp(i, cnt):
            sv = src_v[pl.ds(i * 16, 16)]
            dv = dst_v[pl.ds(i * 16, 16)]
            local = dv - base
            m = (local >= 0) & (local < HALF)
            plsc.store_compressed(src_v.at[pl.ds(cnt, 16)], sv, mask=m)
            plsc.store_compressed(dst_v.at[pl.ds(cnt, 16)], local, mask=m)
            return cnt + jnp.sum(m.astype(jnp.int32))
        cnt = lax.fori_loop(0, NV, comp, jnp.int32(0))

        nchunks = jnp.maximum((cnt + (K - 1)) // K, 1)
        padend = nchunks * K
        iota = lax.iota(jnp.int32, 16)
        w0 = (cnt // 16) * 16
        for t in range(K // 16 + 1):
            w = w0 + t * 16
            pos = w + iota
            inpad = (pos >= cnt) & (pos < padend)
            src_v[pl.ds(w, 16)] = jnp.where(inpad, iota * 8, src_v[pl.ds(w, 16)])
            dst_v[pl.ds(w, 16)] = jnp.where(inpad, DUMP + iota, dst_v[pl.ds(w, 16)])

        plsc.subcore_barrier()

        def gather_start(j, b):
            pltpu.async_copy(
                g_hbm.at[src_v.at[pl.ds(j * K, K)]], rows_v.at[b], sem)

        def gather_wait(j, b):
            pltpu.make_async_copy(
                g_hbm.at[src_v.at[pl.ds(j * K, K)]], rows_v.at[b], sem).wait()

        gather_start(0, jnp.int32(0))

        def body(j, carry):
            b = lax.rem(j, 2)
            nb = lax.rem(j + 1, 2)

            @pl.when(j + 1 < nchunks)
            def _():
                gather_start(j + 1, nb)

            gather_wait(j, b)
            pltpu.sync_copy(rows_v.at[b], acc.at[dst_v.at[pl.ds(j * K, K)]],
                            add=True)
            return carry
        lax.fori_loop(0, nchunks, body, 0)

        plsc.subcore_barrier()
        pltpu.sync_copy(
            acc.at[pl.ds(sid * ROWS_PER_TILE, ROWS_PER_TILE)],
            out_hbm.at[cid, pl.ds(sid * ROWS_PER_TILE, ROWS_PER_TILE)])

    return scat_kernel


R_BLK = 640
GRID = NP // R_BLK
HB = GRID // 2


def _dot(a, b):
    return lax.dot_general(a, b, (((1,), (0,)), ((), ())),
                           precision=lax.Precision.HIGHEST,
                           preferred_element_type=jnp.float32)


def _pre_body(dp_ref, x_ref, w_ref, g_ref, dinv_ref):
    deg = jnp.sum(dp_ref[...], axis=1) + 1.0
    dinv = lax.rsqrt(deg)
    dcol = dinv[:, None]
    g_ref[...] = _dot(x_ref[...], w_ref[...]) * dcol
    dinv_ref[...] = dcol


def _pre_call(degp, X, W0):
    return pl.pallas_call(
        _pre_body,
        grid=(GRID,),
        in_specs=[
            pl.BlockSpec((R_BLK, NW), lambda i: (i, 0)),
            pl.BlockSpec((R_BLK, D), lambda i: (i, 0)),
            pl.BlockSpec((D, D), lambda i: (0, 0)),
        ],
        out_specs=[
            pl.BlockSpec((R_BLK, D), lambda i: (i, 0)),
            pl.BlockSpec((R_BLK, 1), lambda i: (i, 0)),
        ],
        out_shape=[
            jax.ShapeDtypeStruct((NP, D), jnp.float32),
            jax.ShapeDtypeStruct((NP, 1), jnp.float32),
        ],
    )(degp, X, W0)


def _leaky(v):
    return jnp.where(v >= 0.0, v, 0.01 * v)


_S_SPEC = pl.BlockSpec((1, R_BLK, D), lambda i: (i // HB, i % HB, 0))
_ROW_SPEC = pl.BlockSpec((R_BLK, D), lambda i: (i, 0))
_DINV_SPEC = pl.BlockSpec((R_BLK, 1), lambda i: (i, 0))
_ROW_SHAPE = jax.ShapeDtypeStruct((NP, D), jnp.float32)


def _mid_body(rscale, s_ref, g_ref, dinv_ref, x_ref, res_ref, b_ref, w_ref,
              x1_ref, res1_ref, g1_ref):
    dcol = dinv_ref[...]
    conv = (s_ref[0] + g_ref[...]) * dcol + b_ref[...]
    x1 = _leaky(conv) + x_ref[...]
    x1_ref[...] = x1
    res1_ref[...] = res_ref[...] + x1 * rscale
    g1_ref[...] = _dot(x1, w_ref[...]) * dcol


def _mid_call(layer, S, g, dinv, x, res, b, Wn):
    return pl.pallas_call(
        functools.partial(_mid_body, 1.0 / (layer + 2)),
        grid=(GRID,),
        in_specs=[
            _S_SPEC, _ROW_SPEC, _DINV_SPEC, _ROW_SPEC, _ROW_SPEC,
            pl.BlockSpec((1, D), lambda i: (0, 0)),
            pl.BlockSpec((D, D), lambda i: (0, 0)),
        ],
        out_specs=[_ROW_SPEC, _ROW_SPEC, _ROW_SPEC],
        out_shape=[_ROW_SHAPE, _ROW_SHAPE, _ROW_SHAPE],
    )(S, g, dinv, x, res, b, Wn)


def _last_body(rscale, s_ref, g_ref, dinv_ref, x_ref, res_ref, b_ref, res1_ref):
    dcol = dinv_ref[...]
    conv = (s_ref[0] + g_ref[...]) * dcol + b_ref[...]
    x1 = _leaky(conv) + x_ref[...]
    res1_ref[...] = res_ref[...] + x1 * rscale


def _last_call(layer, S, g, dinv, x, res, b):
    return pl.pallas_call(
        functools.partial(_last_body, 1.0 / (layer + 2)),
        grid=(GRID,),
        in_specs=[
            _S_SPEC, _ROW_SPEC, _DINV_SPEC, _ROW_SPEC, _ROW_SPEC,
            pl.BlockSpec((1, D), lambda i: (0, 0)),
        ],
        out_specs=_ROW_SPEC,
        out_shape=_ROW_SHAPE,
    )(S, g, dinv, x, res, b)


def kernel(X, adj_indices, W0, b0, W1, b1, W2, b2):
    src = adj_indices[0].astype(jnp.int32)
    dst = adj_indices[1].astype(jnp.int32)
    E = src.shape[0]

    deg_k = _make_deg_kernel(E)
    scat_k = _make_scatter_kernel(E)

    X_pad = jnp.pad(X, ((0, NP - N_NODES), (0, 0)))

    degp = deg_k(dst)
    degp_t = jnp.pad(jnp.transpose(degp.reshape(NW, N_NODES)),
                     ((0, NP - N_NODES), (0, 0)))
    g0, dinv = _pre_call(degp_t, X_pad, W0)

    S0 = scat_k(g0, src, dst)
    x1, res1, g1 = _mid_call(0, S0, g0, dinv, X_pad, X_pad, b0.reshape(1, D), W1)

    S1 = scat_k(g1, src, dst)
    x2, res2, g2 = _mid_call(1, S1, g1, dinv, x1, res1, b1.reshape(1, D), W2)

    S2 = scat_k(g2, src, dst)
    res3 = _last_call(2, S2, g2, dinv, x2, res2, b2.reshape(1, D))
    return res3[:N_NODES]
